# baseline (speedup 1.0000x reference)
import math

import jax
import jax.numpy as jnp
from jax import lax
from jax.experimental import pallas as pl
from jax.experimental.pallas import tpu as pltpu
from jax.experimental.pallas import tpu_sc as plsc

VOCAB = 1000000
D = 64
NT = 4096
NS_ = 200
B = NT * NS_
SCALE = math.sqrt(D)

_info = plsc.get_sparse_core_info()
NC, NSUB, L = _info.num_cores, _info.num_subcores, _info.num_lanes
NW = NC * NSUB

FULL_BLOCKS = VOCAB // 128
TAIL_W = VOCAB - FULL_BLOCKS * 128
BPW_BASE = FULL_BLOCKS // NW
BPW_EXTRA = FULL_BLOCKS - BPW_BASE * NW


def _repack_body(wt_hbm, wtail_hbm, tab_hbm, blk_v, tb_v):
    wid = lax.axis_index("s") * NC + lax.axis_index("c")
    iota = jax.lax.iota(jnp.int32, 16)

    def transpose_blk():
        def transpose_col(r, _):
            for j0 in range(D // L):
                v = plsc.load_gather(blk_v, [iota + j0 * L, jnp.full((L,), r, jnp.int32)])
                tb_v[r >> 1, pl.ds((r & 1) * D + j0 * L, L)] = v
            return ()

        lax.fori_loop(0, 128, transpose_col, ())

    n_mine = jnp.where(wid < BPW_EXTRA, BPW_BASE + 1, BPW_BASE).astype(jnp.int32)

    def block_step(i, _):
        bl = wid + i * NW
        pltpu.sync_copy(wt_hbm.at[:, pl.ds(pl.multiple_of(bl * 128, 128), 128)], blk_v)
        transpose_blk()
        pltpu.sync_copy(tb_v, tab_hbm.at[pl.ds(pl.multiple_of(bl * 64, 64), 64)])
        return ()

    lax.fori_loop(0, n_mine, block_step, ())

    @pl.when(wid == NW - 1)
    def _():
        pltpu.sync_copy(wtail_hbm, blk_v)
        transpose_blk()
        pltpu.sync_copy(tb_v.at[pl.ds(0, 32)], tab_hbm.at[pl.ds(VOCAB // 2 - 32, 32)])


N_CHUNKS = B // 128
CPW = N_CHUNKS // NW


def _lookup_body(xt_hbm, tab_hbm, out_hbm, idx_v, g_v, tb_v, sem):
    wid = lax.axis_index("s") * NC + lax.axis_index("c")
    iota = jax.lax.iota(jnp.int32, 16)

    def chunk_step(i, _):
        c = wid * CPW + i
        t = c // 32
        bc = c % 32
        pltpu.sync_copy(xt_hbm.at[t, pl.ds(pl.multiple_of(bc * 128, 128), 128)], idx_v.at[0])
        for k in range(8):
            sl = pl.ds(k * L, L)
            v = idx_v[0, sl]
            idx_v[1, sl] = jax.lax.shift_right_logical(v, 1)
        pltpu.async_copy(tab_hbm.at[idx_v.at[1]], g_v, sem).wait()
        for bl0 in range(8):
            pv = (idx_v[0, pl.ds(bl0 * L, L)] & 1) * D
            rows = iota + bl0 * L

            def emit_j(j, _):
                v = plsc.load_gather(g_v, [rows, pv + j]) * SCALE
                tb_v[j >> 3, j & 7, pl.ds(bl0 * L, L)] = v
                return ()

            lax.fori_loop(0, D, emit_j, ())
        pltpu.sync_copy(tb_v, out_hbm.at[t, :, bc])
        return ()

    lax.fori_loop(0, CPW, chunk_step, ())


@jax.jit
def _embed(xt, wt, wtail):
    mesh = plsc.VectorSubcoreMesh(core_axis_name="c", subcore_axis_name="s")
    repack = pl.kernel(
        _repack_body,
        out_type=jax.ShapeDtypeStruct((VOCAB // 2, 128), jnp.float32),
        mesh=mesh,
        scratch_types=[
            pltpu.VMEM((D, 128), jnp.float32),
            pltpu.VMEM((D, 128), jnp.float32),
        ],
        compiler_params=pltpu.CompilerParams(use_tc_tiling_on_sc=True, needs_layout_passes=False),
    )
    tab = repack(wt, wtail)
    lookup = pl.kernel(
        _lookup_body,
        out_type=jax.ShapeDtypeStruct((NS_, 8, 32, 8, 128), jnp.float32),
        mesh=mesh,
        scratch_types=[
            pltpu.VMEM((2, 128), jnp.int32),
            pltpu.VMEM((128, 128), jnp.float32),
            pltpu.VMEM((8, 8, 128), jnp.float32),
            pltpu.SemaphoreType.DMA,
        ],
        compiler_params=pltpu.CompilerParams(use_tc_tiling_on_sc=True, needs_layout_passes=False),
    )
    return lookup(xt, tab)


def kernel(x, embed_weight):
    xt = x.astype(jnp.int32).T
    wt = embed_weight.T
    wtail = jnp.pad(embed_weight[VOCAB - 64:].T, ((0, 0), (0, 64)))
    out5 = _embed(xt, wt, wtail)
    return out5.transpose(2, 4, 0, 1, 3).reshape(NT, NS_, D)


# speedup vs baseline: 1.2715x; 1.2715x over previous
import math

import jax
import jax.numpy as jnp
from jax import lax
from jax.experimental import pallas as pl
from jax.experimental.pallas import tpu as pltpu
from jax.experimental.pallas import tpu_sc as plsc

VOCAB = 1000000
D = 64
NT = 4096
NS_ = 200
B = NT * NS_
SCALE = math.sqrt(D)

_info = plsc.get_sparse_core_info()
NC, NSUB, L = _info.num_cores, _info.num_subcores, _info.num_lanes
NW = NC * NSUB

FULL_BLOCKS = VOCAB // 128
BPW_BASE = FULL_BLOCKS // NW
BPW_EXTRA = FULL_BLOCKS - BPW_BASE * NW


def _repack_body(wt_hbm, wtail_hbm, tab_hbm, blk_v, tb_v, semg, sems):
    wid = lax.axis_index("s") * NC + lax.axis_index("c")
    iota = jax.lax.iota(jnp.int32, L)

    def fire_load(i, b):
        bl = wid + i * NW
        pltpu.async_copy(
            wt_hbm.at[:, pl.ds(pl.multiple_of(bl * 128, 128), 128)],
            blk_v.at[b], semg.at[b],
        )

    def transpose_blk(b):
        def transpose_pair(r2, _):
            for rr in range(2):
                rv = jnp.full((L,), r2 * 2 + rr, jnp.int32)
                for j0 in range(D // L):
                    v = plsc.load_gather(blk_v.at[b], [iota + j0 * L, rv])
                    tb_v[b, r2, pl.ds(rr * D + j0 * L, L)] = v
            return ()

        lax.fori_loop(0, 64, transpose_pair, ())

    n_mine = jnp.where(wid < BPW_EXTRA, BPW_BASE + 1, BPW_BASE).astype(jnp.int32)

    fire_load(0, 0)

    def block_step(i, _):
        b = i & 1
        bl = wid + i * NW

        @pl.when(i + 1 < n_mine)
        def _():
            fire_load(i + 1, 1 - b)

        pltpu.make_async_copy(
            wt_hbm.at[:, pl.ds(0, 128)], blk_v.at[b], semg.at[b]
        ).wait()

        @pl.when(i >= 2)
        def _():
            pltpu.make_async_copy(
                tab_hbm.at[pl.ds(0, 64)], tb_v.at[b], sems.at[b]
            ).wait()

        transpose_blk(b)
        pltpu.async_copy(
            tb_v.at[b], tab_hbm.at[pl.ds(pl.multiple_of(bl * 64, 64), 64)],
            sems.at[b],
        )
        return ()

    lax.fori_loop(0, n_mine, block_step, ())

    for b in range(2):
        pltpu.make_async_copy(
            tab_hbm.at[pl.ds(0, 64)], tb_v.at[b], sems.at[b]
        ).wait()

    @pl.when(wid == NW - 1)
    def _():
        pltpu.sync_copy(wtail_hbm, blk_v.at[0])
        transpose_blk(0)
        pltpu.sync_copy(tb_v.at[0, pl.ds(0, 32)], tab_hbm.at[pl.ds(VOCAB // 2 - 32, 32)])


N_CHUNKS = B // 128
CPW = N_CHUNKS // NW


def _lookup_body(xt_hbm, tab_hbm, out_hbm, idx_v, g_v, tb_v, semi, semg, sems):
    wid = lax.axis_index("s") * NC + lax.axis_index("c")
    iota = jax.lax.iota(jnp.int32, L)
    c0 = wid * CPW

    def fire_idx(i, b):
        c = c0 + i
        pltpu.async_copy(
            xt_hbm.at[c // 32, pl.ds(pl.multiple_of((c % 32) * 128, 128), 128)],
            idx_v.at[b], semi.at[b],
        )

    fire_idx(0, 0)

    def step(i, _):
        b = i & 1
        p = (i - 1) & 1

        @pl.when(i < CPW)
        def _():
            pltpu.make_async_copy(
                xt_hbm.at[0, pl.ds(0, 128)], idx_v.at[b], semi.at[b]
            ).wait()
            pltpu.async_copy(tab_hbm.at[idx_v.at[b]], g_v.at[b], semg.at[b])

        @pl.when(i >= 1)
        def _():
            pltpu.make_async_copy(
                tab_hbm.at[pl.ds(0, 128)], g_v.at[p], semg.at[p]
            ).wait()

        @pl.when(i + 1 < CPW)
        def _():
            fire_idx(i + 1, p)

        @pl.when(i >= 1)
        def _():
            c = c0 + i - 1

            @pl.when(i - 1 >= 2)
            def _():
                pltpu.make_async_copy(
                    out_hbm.at[0, :, 0], tb_v.at[p], sems.at[p]
                ).wait()

            for bl0 in range(8):
                rows = iota + bl0 * L

                def emit_jg(jg, _):
                    for jr in range(8):
                        jv = jnp.full((L,), jg * 8 + jr, jnp.int32)
                        v = plsc.load_gather(g_v.at[p], [rows, jv]) * SCALE
                        tb_v[p, jg, jr, pl.ds(bl0 * L, L)] = v
                    return ()

                lax.fori_loop(0, 8, emit_jg, ())
            pltpu.async_copy(tb_v.at[p], out_hbm.at[c // 32, :, c % 32], sems.at[p])

        return ()

    lax.fori_loop(0, CPW + 1, step, ())

    for b in range(2):
        pltpu.make_async_copy(
            out_hbm.at[0, :, 0], tb_v.at[b], sems.at[b]
        ).wait()


@jax.jit
def _embed(xt, wt, wtail):
    mesh = plsc.VectorSubcoreMesh(core_axis_name="c", subcore_axis_name="s")
    repack = pl.kernel(
        _repack_body,
        out_type=jax.ShapeDtypeStruct((VOCAB // 2, 128), jnp.float32),
        mesh=mesh,
        scratch_types=[
            pltpu.VMEM((2, D, 128), jnp.float32),
            pltpu.VMEM((2, D, 128), jnp.float32),
            pltpu.SemaphoreType.DMA((2,)),
            pltpu.SemaphoreType.DMA((2,)),
        ],
        compiler_params=pltpu.CompilerParams(use_tc_tiling_on_sc=True, needs_layout_passes=False),
    )
    tab = repack(wt, wtail)
    tabl = tab.reshape(VOCAB, D)
    lookup = pl.kernel(
        _lookup_body,
        out_type=jax.ShapeDtypeStruct((NS_, 8, 32, 8, 128), jnp.float32),
        mesh=mesh,
        scratch_types=[
            pltpu.VMEM((2, 128), jnp.int32),
            pltpu.VMEM((2, 128, D), jnp.float32),
            pltpu.VMEM((2, 8, 8, 128), jnp.float32),
            pltpu.SemaphoreType.DMA((2,)),
            pltpu.SemaphoreType.DMA((2,)),
            pltpu.SemaphoreType.DMA((2,)),
        ],
        compiler_params=pltpu.CompilerParams(use_tc_tiling_on_sc=False, needs_layout_passes=False),
    )
    return lookup(xt, tabl)


def kernel(x, embed_weight):
    xt = x.astype(jnp.int32).T
    wt = embed_weight.T
    wtail = jnp.pad(embed_weight[VOCAB - 64:].T, ((0, 0), (0, 64)))
    out5 = _embed(xt, wt, wtail)
    return out5.transpose(2, 4, 0, 1, 3).reshape(NT, NS_, D)


# speedup vs baseline: 2.3685x; 1.8628x over previous
import math

import jax
import jax.numpy as jnp
from jax import lax
from jax.experimental import pallas as pl
from jax.experimental.pallas import tpu as pltpu
from jax.experimental.pallas import tpu_sc as plsc

VOCAB = 1000000
D = 64
NT = 4096
NS_ = 200
B = NT * NS_
SCALE = math.sqrt(D)

_info = plsc.get_sparse_core_info()
NC, NSUB, L = _info.num_cores, _info.num_subcores, _info.num_lanes
NW = NC * NSUB

FULL_BLOCKS = VOCAB // 128
BPW_BASE = FULL_BLOCKS // NW
BPW_EXTRA = FULL_BLOCKS - BPW_BASE * NW


def _repack_body(wt_hbm, wtail_hbm, tab_hbm, blk_v, tb_v, semg, sems):
    wid = lax.axis_index("s") * NC + lax.axis_index("c")
    iota = jax.lax.iota(jnp.int32, L)

    def fire_load(i, b):
        bl = wid + i * NW
        pltpu.async_copy(
            wt_hbm.at[:, pl.ds(pl.multiple_of(bl * 128, 128), 128)],
            blk_v.at[b], semg.at[b],
        )

    def transpose_blk(b):
        @plsc.parallel_loop(0, 64, unroll=4)
        def transpose_pair(r2):
            for rr in range(2):
                rv = jnp.full((L,), r2 * 2 + rr, jnp.int32)
                for j0 in range(D // L):
                    v = plsc.load_gather(blk_v.at[b], [iota + j0 * L, rv])
                    tb_v[b, r2, pl.ds(rr * D + j0 * L, L)] = v

    n_mine = jnp.where(wid < BPW_EXTRA, BPW_BASE + 1, BPW_BASE).astype(jnp.int32)

    fire_load(0, 0)

    def block_step(i, _):
        b = i & 1
        bl = wid + i * NW

        @pl.when(i + 1 < n_mine)
        def _():
            fire_load(i + 1, 1 - b)

        pltpu.make_async_copy(
            wt_hbm.at[:, pl.ds(0, 128)], blk_v.at[b], semg.at[b]
        ).wait()

        @pl.when(i >= 2)
        def _():
            pltpu.make_async_copy(
                tab_hbm.at[pl.ds(0, 64)], tb_v.at[b], sems.at[b]
            ).wait()

        transpose_blk(b)
        pltpu.async_copy(
            tb_v.at[b], tab_hbm.at[pl.ds(pl.multiple_of(bl * 64, 64), 64)],
            sems.at[b],
        )
        return ()

    lax.fori_loop(0, n_mine, block_step, ())

    for b in range(2):
        pltpu.make_async_copy(
            tab_hbm.at[pl.ds(0, 64)], tb_v.at[b], sems.at[b]
        ).wait()

    @pl.when(wid == NW - 1)
    def _():
        pltpu.sync_copy(wtail_hbm, blk_v.at[0])
        transpose_blk(0)
        pltpu.sync_copy(tb_v.at[0, pl.ds(0, 32)], tab_hbm.at[pl.ds(VOCAB // 2 - 32, 32)])


N_CHUNKS = B // 128
CPW = N_CHUNKS // NW


def _lookup_body(xt_hbm, tab_hbm, out_hbm, idx_v, g_v, tb_v, semi, semg, sems):
    wid = lax.axis_index("s") * NC + lax.axis_index("c")
    iota = jax.lax.iota(jnp.int32, L)
    c0 = wid * CPW

    def fire_idx(i, b):
        c = c0 + i
        pltpu.async_copy(
            xt_hbm.at[c // 32, pl.ds(pl.multiple_of((c % 32) * 128, 128), 128)],
            idx_v.at[b], semi.at[b],
        )

    fire_idx(0, 0)

    def step(i, _):
        b = i & 1
        p = (i - 1) & 1

        @pl.when(i < CPW)
        def _():
            pltpu.make_async_copy(
                xt_hbm.at[0, pl.ds(0, 128)], idx_v.at[b], semi.at[b]
            ).wait()
            pltpu.async_copy(tab_hbm.at[idx_v.at[b]], g_v.at[b], semg.at[b])

        @pl.when(i >= 1)
        def _():
            pltpu.make_async_copy(
                tab_hbm.at[pl.ds(0, 128)], g_v.at[p], semg.at[p]
            ).wait()

        @pl.when(i + 1 < CPW)
        def _():
            fire_idx(i + 1, p)

        @pl.when(i >= 1)
        def _():
            c = c0 + i - 1

            @pl.when(i - 1 >= 2)
            def _():
                pltpu.make_async_copy(
                    out_hbm.at[0, :, 0], tb_v.at[p], sems.at[p]
                ).wait()

            for bl0 in range(8):
                rows = iota + bl0 * L

                @plsc.parallel_loop(0, 8, unroll=2)
                def emit_jg(jg):
                    for jr in range(8):
                        jv = jnp.full((L,), jg * 8 + jr, jnp.int32)
                        v = plsc.load_gather(g_v.at[p], [rows, jv]) * SCALE
                        tb_v[p, jg, jr, pl.ds(bl0 * L, L)] = v
            pltpu.async_copy(tb_v.at[p], out_hbm.at[c // 32, :, c % 32], sems.at[p])

        return ()

    lax.fori_loop(0, CPW + 1, step, ())

    for b in range(2):
        pltpu.make_async_copy(
            out_hbm.at[0, :, 0], tb_v.at[b], sems.at[b]
        ).wait()


@jax.jit
def _embed(xt, wt, wtail):
    mesh = plsc.VectorSubcoreMesh(core_axis_name="c", subcore_axis_name="s")
    repack = pl.kernel(
        _repack_body,
        out_type=jax.ShapeDtypeStruct((VOCAB // 2, 128), jnp.float32),
        mesh=mesh,
        scratch_types=[
            pltpu.VMEM((2, D, 128), jnp.float32),
            pltpu.VMEM((2, D, 128), jnp.float32),
            pltpu.SemaphoreType.DMA((2,)),
            pltpu.SemaphoreType.DMA((2,)),
        ],
        compiler_params=pltpu.CompilerParams(use_tc_tiling_on_sc=True, needs_layout_passes=False),
    )
    tab = repack(wt, wtail)
    tabl = tab.reshape(VOCAB, D)
    lookup = pl.kernel(
        _lookup_body,
        out_type=jax.ShapeDtypeStruct((NS_, 8, 32, 8, 128), jnp.float32),
        mesh=mesh,
        scratch_types=[
            pltpu.VMEM((2, 128), jnp.int32),
            pltpu.VMEM((2, 128, D), jnp.float32),
            pltpu.VMEM((2, 8, 8, 128), jnp.float32),
            pltpu.SemaphoreType.DMA((2,)),
            pltpu.SemaphoreType.DMA((2,)),
            pltpu.SemaphoreType.DMA((2,)),
        ],
        compiler_params=pltpu.CompilerParams(use_tc_tiling_on_sc=False, needs_layout_passes=False),
    )
    return lookup(xt, tabl)


def kernel(x, embed_weight):
    xt = x.astype(jnp.int32).T
    wt = embed_weight.T
    wtail = jnp.pad(embed_weight[VOCAB - 64:].T, ((0, 0), (0, 64)))
    out5 = _embed(xt, wt, wtail)
    return out5.transpose(2, 4, 0, 1, 3).reshape(NT, NS_, D)
